# baseline (speedup 1.0000x reference)
import jax
import jax.numpy as jnp
from jax.experimental import pallas as pl

N = 1024
_NEG = -1e30


def _layer(h_in, W, a_src, a_dst, b, valid):
    h = jnp.dot(h_in, W, preferred_element_type=jnp.float32)
    s = jnp.sum(h * a_src, axis=1)
    d = jnp.sum(h * a_dst, axis=1)
    e = s[:, None] + d[None, :]
    e = jnp.where(e >= 0.0, e, 0.2 * e)
    e = jnp.where(valid, e, _NEG)
    w = jnp.exp(e)
    den = jnp.sum(w, axis=0)
    coef = w * (1.0 / (den + 1e-16))[None, :]
    out = jax.lax.dot_general(
        coef.astype(jnp.bfloat16), h.astype(jnp.bfloat16),
        (((0,), (0,)), ((), ())), preferred_element_type=jnp.float32
    )
    return out + b


def _gat2_kernel(
    x_ref, adj_ref, w1_ref, as1_ref, ad1_ref, b1_ref,
    w2_ref, as2_ref, ad2_ref, b2_ref, out_ref,
):
    adj = adj_ref[...]
    row = jax.lax.broadcasted_iota(jnp.int32, (N, N), 0)
    col = jax.lax.broadcasted_iota(jnp.int32, (N, N), 1)
    valid = jnp.logical_or(row == col, adj != 0)

    h1 = _layer(x_ref[...], w1_ref[...], as1_ref[...], ad1_ref[...],
                b1_ref[...], valid)
    h1 = jnp.maximum(h1, 0.0)
    out_ref[...] = _layer(h1, w2_ref[...], as2_ref[...], ad2_ref[...],
                          b2_ref[...], valid)


def kernel(x, adj, W1, att_src1, att_dst1, b1, W2, att_src2, att_dst2, b2):
    fout = W2.shape[1]
    return pl.pallas_call(
        _gat2_kernel,
        out_shape=jax.ShapeDtypeStruct((N, fout), jnp.float32),
    )(
        x, adj,
        W1, att_src1[None, :], att_dst1[None, :], b1[None, :],
        W2, att_src2[None, :], att_dst2[None, :], b2[None, :],
    )


# speedup vs baseline: 1.0483x; 1.0483x over previous
import jax
import jax.numpy as jnp
from jax.experimental import pallas as pl

N = 1024
_NEG = -1e30


def _layer(h_in, W, a_src, a_dst, b, mask_add):
    h = jnp.dot(h_in, W, preferred_element_type=jnp.float32)
    s = jnp.sum(h * a_src, axis=1)
    d = jnp.sum(h * a_dst, axis=1)
    e = s[:, None] + d[None, :] + mask_add
    e = jnp.maximum(e, 0.2 * e)
    w = jnp.exp(e)
    den = jnp.sum(w, axis=0)
    coef = w * (1.0 / (den + 1e-16))[None, :]
    out = jax.lax.dot_general(
        coef.astype(jnp.bfloat16), h.astype(jnp.bfloat16),
        (((0,), (0,)), ((), ())), preferred_element_type=jnp.float32
    )
    return out + b


def _gat2_kernel(
    x_ref, adj_ref, w1_ref, as1_ref, ad1_ref, b1_ref,
    w2_ref, as2_ref, ad2_ref, b2_ref, out_ref,
):
    adj = adj_ref[...]
    row = jax.lax.broadcasted_iota(jnp.int32, (N, N), 0)
    col = jax.lax.broadcasted_iota(jnp.int32, (N, N), 1)
    valid = jnp.logical_or(row == col, adj != 0)
    mask_add = jnp.where(valid, 0.0, _NEG).astype(jnp.float32)

    h1 = _layer(x_ref[...], w1_ref[...], as1_ref[...], ad1_ref[...],
                b1_ref[...], mask_add)
    h1 = jnp.maximum(h1, 0.0)
    out_ref[...] = _layer(h1, w2_ref[...], as2_ref[...], ad2_ref[...],
                          b2_ref[...], mask_add)


def kernel(x, adj, W1, att_src1, att_dst1, b1, W2, att_src2, att_dst2, b2):
    fout = W2.shape[1]
    return pl.pallas_call(
        _gat2_kernel,
        out_shape=jax.ShapeDtypeStruct((N, fout), jnp.float32),
    )(
        x, adj,
        W1, att_src1[None, :], att_dst1[None, :], b1[None, :],
        W2, att_src2[None, :], att_dst2[None, :], b2[None, :],
    )
